# baseline (speedup 1.0000x reference)
import jax
import jax.numpy as jnp
from jax import lax
from jax.experimental import pallas as pl
from jax.experimental.pallas import tpu as pltpu
from jax.experimental.pallas import tpu_sc as plsc

N = 10000
D = 128
E = 320000
C = 32
B = 250000
K = 1000
M_ATT = 1000
EPS = 1e-7
LR = 1000.0

NC = 2
NS = 16
NW = NC * NS
SH = 7824
BP = NW * SH
ECH = 2000
BCH = 2608
GCH = 128
GCA = 256
TRASH = N
NROWS = 10008
IPAD = 1008
CNTSZ = 10016
CSZ = 10384
NCH_MAX = 81
OUTP = 1152


def _mesh():
    return plsc.VectorSubcoreMesh(core_axis_name="c", subcore_axis_name="s")


def _wid():
    return lax.axis_index("s") * NC + lax.axis_index("c")


def _idiv(a, b):
    return lax.div(a, jnp.int32(b))


def _irem(a, b):
    return lax.rem(a, jnp.int32(b))


def _popcnt(m):
    return plsc.all_reduce_population_count(m)[0]


def _memset_i32(ref, size, value):
    v = jnp.full((16,), value, jnp.int32)

    def body(i, carry):
        ref[pl.ds(i * 16, 16)] = v
        return carry

    lax.fori_loop(jnp.int32(0), size // 16, body, jnp.int32(0))


def _memset_f32(ref, size, value):
    v = jnp.full((16,), value, jnp.float32)

    def body(i, carry):
        ref[pl.ds(i * 16, 16)] = v
        return carry

    lax.fori_loop(jnp.int32(0), size // 16, body, jnp.int32(0))


def _build_cnt(cnt_ref, att_ref):
    _memset_i32(cnt_ref, CNTSZ, 0)
    ones = jnp.ones((16,), jnp.int32)

    def body(i, carry):
        idx = att_ref[pl.ds(i * 16, 16)]
        plsc.addupdate_scatter(cnt_ref, [idx], ones)
        return carry

    lax.fori_loop(jnp.int32(0), IPAD // 16, body, jnp.int32(0))


def _ka_body(x0_hbm, x1_hbm, src_hbm, dst_hbm, blk_hbm, att_hbm,
             agg_hbm, deg_hbm, cnt_hbm,
             cnt_v, deg_v, att_v, sbuf_a, sbuf_b,
             csrc_v, cdst_v, rows_v, rows2_v, agg_sh, sem, sem2):
    wid = _wid()
    cid = lax.axis_index("c")
    sid = lax.axis_index("s")

    pltpu.sync_copy(att_hbm, att_v)
    _build_cnt(cnt_v, att_v)
    _memset_f32(deg_v, CNTSZ, 0.0)

    def zero_rows():
        def zz_body(i, carry):
            rows_v[_idiv(i, 4), pl.ds(_irem(i, 4) * 16, 16)] = \
                jnp.zeros((16,), jnp.float32)
            return carry

        lax.fori_loop(jnp.int32(0), GCA * (D // 32), zz_body, jnp.int32(0))

    r0 = sid * 640
    starts = [0, GCA, 640 - GCA]

    def sp_start(j):
        s = r0 + starts[j]
        return pl.multiple_of(jnp.minimum(s, NROWS - GCA), 8)

    def zero_agg():
        zero_rows()
        for j in range(len(starts)):
            pltpu.sync_copy(rows_v, agg_sh.at[pl.ds(sp_start(j), GCA)])

    def process_edges(n_edges, ech, base_edge, is_block, half, do_deg):
        _memset_i32(csrc_v, CSZ, 0)
        _memset_i32(cdst_v, CSZ, TRASH)

        def stage_body(s, nsurv):
            sb = pl.multiple_of(base_edge + s * ech, 16)
            if is_block:
                pltpu.sync_copy(blk_hbm.at[pl.ds(sb, ech)],
                                sbuf_a.at[pl.ds(0, ech)])
            else:
                pltpu.sync_copy(src_hbm.at[pl.ds(sb, ech)],
                                sbuf_a.at[pl.ds(0, ech)])
                pltpu.sync_copy(dst_hbm.at[pl.ds(sb, ech)],
                                sbuf_b.at[pl.ds(0, ech)])

            def grp_body(k, off):
                if is_block:
                    b16 = sbuf_a[pl.ds(k * 16, 16)]
                    r16 = _idiv(b16, N)
                    c16 = _irem(b16, N)
                    c16 = jnp.where(r16 == c16, _irem(c16 + 1, N), c16)
                    gid = sb + k * 16 + lax.iota(jnp.int32, 16)
                    valid = gid < B
                    s16, d16 = r16, c16
                    w16 = jnp.where(valid, jnp.float32(EPS), 0.0)
                else:
                    s16 = sbuf_a[pl.ds(k * 16, 16)]
                    d16 = sbuf_b[pl.ds(k * 16, 16)]
                    valid = d16 >= 0
                    w16 = jnp.ones((16,), jnp.float32)
                if do_deg:
                    plsc.addupdate_scatter(deg_v, [d16], w16)
                m = (plsc.load_gather(cnt_v, [d16]) > 0) & valid
                plsc.store_compressed(csrc_v.at[pl.ds(off, 16)], s16, mask=m)
                plsc.store_compressed(cdst_v.at[pl.ds(off, 16)], d16, mask=m)
                return off + _popcnt(m)

            return lax.fori_loop(jnp.int32(0), ech // 16, grp_body, nsurv)

        nsurv = lax.fori_loop(jnp.int32(0), n_edges // ech, stage_body, jnp.int32(0))

        nch = _idiv(nsurv + GCA - 1, GCA)

        xh_hbm = x1_hbm if half else x0_hbm
        bufs = (rows_v, rows2_v)
        sems = (sem, sem2)

        def fire(r, b):
            rg = pl.multiple_of(r * GCA, 8)
            return pltpu.async_copy(
                xh_hbm.at[csrc_v.at[pl.ds(rg, GCA)]], bufs[b], sems[b])

        def drain(r, b):
            rg = pl.multiple_of(r * GCA, 8)
            buf = bufs[b]
            if is_block:
                def sc_body(q, c2):
                    qr = _idiv(q, 4)
                    qc = _irem(q, 4) * 16
                    v = buf[qr, pl.ds(qc, 16)]
                    buf[qr, pl.ds(qc, 16)] = v * jnp.float32(EPS)
                    return c2

                lax.fori_loop(jnp.int32(0), GCA * (D // 32), sc_body,
                              jnp.int32(0))
            descs = []
            for sub in range(GCA // 16):
                idx16 = cdst_v[pl.ds(rg + sub * 16, 16)]
                descs.append(pltpu.async_copy(
                    buf.at[pl.ds(sub * 16, 16)], agg_sh.at[idx16],
                    sems[b], add=True))
            for dsc in descs:
                dsc.wait()

        def wait_g(r, b):
            rg = pl.multiple_of(r * GCA, 8)
            pltpu.make_async_copy(
                xh_hbm.at[csrc_v.at[pl.ds(rg, GCA)]], bufs[b],
                sems[b]).wait()

        @pl.when(nch > 0)
        def _():
            fire(jnp.int32(0), 0)

        def pair_body(pp, carry):
            r0 = pp * 2
            r1 = r0 + 1

            @pl.when(r1 < nch)
            def _():
                fire(r1, 1)

            wait_g(r0, 0)
            drain(r0, 0)

            @pl.when(r0 + 2 < nch)
            def _():
                fire(r0 + 2, 0)

            @pl.when(r1 < nch)
            def _():
                wait_g(r1, 1)
                drain(r1, 1)
            return carry

        lax.fori_loop(jnp.int32(0), _idiv(nch + 1, 2), pair_body,
                      jnp.int32(0))

    for h in range(2):
        zero_agg()
        plsc.subcore_barrier()
        process_edges(E // NW, ECH, wid * (E // NW), False, h, h == 0)
        process_edges(SH, BCH, wid * SH, True, h, h == 0)
        plsc.subcore_barrier()
        for j in range(len(starts)):
            sj = sp_start(j)
            pltpu.sync_copy(agg_sh.at[pl.ds(sj, GCA)], rows_v)
            pltpu.sync_copy(rows_v,
                            agg_hbm.at[cid, jnp.int32(h), pl.ds(sj, GCA)])
        plsc.subcore_barrier()

    pltpu.sync_copy(deg_v.at[pl.ds(0, N)],
                    deg_hbm.at[pl.ds(pl.multiple_of(wid * N, 8), N)])

    @pl.when(wid == 0)
    def _():
        pltpu.sync_copy(cnt_v.at[pl.ds(0, N)], cnt_hbm)


def _run_ka(x0, x1, src, dst, blk, att):
    f = pl.kernel(
        _ka_body,
        out_type=(
            jax.ShapeDtypeStruct((NC, 2, NROWS, D // 2), jnp.float32),
            jax.ShapeDtypeStruct((NW * N,), jnp.float32),
            jax.ShapeDtypeStruct((N,), jnp.int32),
        ),
        mesh=_mesh(),
        compiler_params=pltpu.CompilerParams(needs_layout_passes=False,
                                             use_tc_tiling_on_sc=False),
        scratch_types=[
            pltpu.VMEM((CNTSZ,), jnp.int32),
            pltpu.VMEM((CNTSZ,), jnp.float32),
            pltpu.VMEM((IPAD,), jnp.int32),
            pltpu.VMEM((BCH,), jnp.int32),
            pltpu.VMEM((BCH,), jnp.int32),
            pltpu.VMEM((CSZ,), jnp.int32),
            pltpu.VMEM((CSZ,), jnp.int32),
            pltpu.VMEM((GCA, D // 2), jnp.float32),
            pltpu.VMEM((GCA, D // 2), jnp.float32),
            pltpu.VMEM_SHARED((NROWS, D // 2), jnp.float32),
            pltpu.SemaphoreType.DMA,
            pltpu.SemaphoreType.DMA,
        ],
    )
    return f(x0, x1, src, dst, blk, att)


def _kb_body(x_ref, agg_ref, deg_ref, cntf_ref, lab_ref, w_ref, wt_ref,
             u_ref, s2_ref):
    x = x_ref[...]
    agg = jnp.concatenate(
        [agg_ref[0, 0] + agg_ref[1, 0], agg_ref[0, 1] + agg_ref[1, 1]],
        axis=-1)
    deg = jnp.sum(deg_ref[...], axis=1) + 1.0
    h = (agg + x) / deg[:, None]
    logits = jnp.dot(h, w_ref[...], preferred_element_type=jnp.float32)
    mx = jnp.max(logits, axis=-1, keepdims=True)
    ex = jnp.exp(logits - mx)
    p = ex / jnp.sum(ex, axis=-1, keepdims=True)
    lab = lab_ref[...]
    oh = lab == lax.broadcasted_iota(jnp.int32, (1, C), 1)
    pm = jnp.where(oh, -jnp.inf, p)
    bo = jnp.max(pm, axis=-1, keepdims=True)
    bsel = jnp.logical_and(jnp.logical_not(oh), p == bo)
    nb = jnp.maximum(
        jnp.sum(bsel.astype(jnp.float32), axis=-1, keepdims=True), 1.0)
    cnt = cntf_ref[...]
    dl_p = (cnt / jnp.float32(M_ATT)) * (bsel.astype(jnp.float32) / nb
                                         - oh.astype(jnp.float32))
    dlg = p * (dl_p - jnp.sum(dl_p * p, axis=-1, keepdims=True))
    dh = jnp.dot(dlg, wt_ref[...], preferred_element_type=jnp.float32)
    u = dh / deg[:, None]
    u_ref[...] = u
    s2_ref[...] = jnp.sum(u * h, axis=-1, keepdims=True)


def _run_kb(x, agg2, deg32, cntf, lab, W, WT):
    blk = 1000
    z = lambda i: i * 0
    return pl.pallas_call(
        _kb_body,
        grid=(N // blk,),
        in_specs=[
            pl.BlockSpec((blk, D), lambda i: (i, z(i))),
            pl.BlockSpec((NC, 2, blk, D // 2),
                         lambda i: (z(i), z(i), i, z(i))),
            pl.BlockSpec((blk, NW), lambda i: (i, z(i))),
            pl.BlockSpec((blk, 1), lambda i: (i, z(i))),
            pl.BlockSpec((blk, 1), lambda i: (i, z(i))),
            pl.BlockSpec((D, C), lambda i: (z(i), z(i))),
            pl.BlockSpec((C, D), lambda i: (z(i), z(i))),
        ],
        out_specs=[
            pl.BlockSpec((blk, D), lambda i: (i, z(i))),
            pl.BlockSpec((blk, 1), lambda i: (i, z(i))),
        ],
        out_shape=[
            jax.ShapeDtypeStruct((N, D), jnp.float32),
            jax.ShapeDtypeStruct((N, 1), jnp.float32),
        ],
    )(x, agg2, deg32, cntf, lab, W, WT)


def _kc_body(u_hbm, x_hbm, blk_hbm, s2_hbm, att_hbm, lr_hbm,
             w1_hbm,
             cnt_v, s2_v, att_v, lr_v, bbuf,
             cu_v, cx_v, cp_v, ubuf, xbuf, w1_v, sem, sem2):
    wid = _wid()
    base = pl.multiple_of(wid * SH, 16)

    pltpu.sync_copy(att_hbm, att_v)
    _build_cnt(cnt_v, att_v)
    pltpu.sync_copy(s2_hbm, s2_v.at[pl.ds(0, N)])
    pltpu.sync_copy(lr_hbm, lr_v)
    lr16 = lr_v[...]

    def init_body(k, carry):
        gid = base + k * 16 + lax.iota(jnp.int32, 16)
        w1_v[pl.ds(k * 16, 16)] = jnp.where(gid < B, jnp.float32(EPS), -1.0)
        return carry

    lax.fori_loop(jnp.int32(0), SH // 16, init_body, jnp.int32(0))

    _memset_i32(cu_v, CSZ, 0)
    _memset_i32(cx_v, CSZ, 0)
    _memset_i32(cp_v, CSZ, SH)

    def stage_body(s, nsurv):
        sb = s * BCH
        bs = pl.multiple_of(base + sb, 16)
        pltpu.sync_copy(blk_hbm.at[pl.ds(bs, BCH)], bbuf)

        def grp_body(k, off):
            b16 = bbuf[pl.ds(k * 16, 16)]
            r16 = _idiv(b16, N)
            c16 = _irem(b16, N)
            c16 = jnp.where(r16 == c16, _irem(c16 + 1, N), c16)
            lpos = sb + k * 16 + lax.iota(jnp.int32, 16)
            gid = base + lpos
            m = (plsc.load_gather(cnt_v, [c16]) > 0) & (gid < B)
            plsc.store_compressed(cu_v.at[pl.ds(off, 16)], c16, mask=m)
            plsc.store_compressed(cx_v.at[pl.ds(off, 16)], r16, mask=m)
            plsc.store_compressed(cp_v.at[pl.ds(off, 16)], lpos, mask=m)
            return off + _popcnt(m)

        return lax.fori_loop(jnp.int32(0), BCH // 16, grp_body, nsurv)

    nsurv = lax.fori_loop(jnp.int32(0), SH // BCH, stage_body, jnp.int32(0))

    nch = _idiv(nsurv + GCH - 1, GCH)
    lane16 = lax.iota(jnp.int32, 16)

    def gs_body(r, carry):
        rg = pl.multiple_of(r * GCH, 8)
        du = pltpu.async_copy(u_hbm.at[cu_v.at[pl.ds(rg, GCH)]], ubuf, sem)
        dx = pltpu.async_copy(x_hbm.at[cx_v.at[pl.ds(rg, GCH)]], xbuf, sem2)
        du.wait()
        dx.wait()

        def q_body(q, c2):
            row16 = q * 16 + lane16
            acc = jnp.zeros((16,), jnp.float32)

            def d_body(d, a):
                d16 = jnp.full((16,), 0, jnp.int32) + d
                uu = plsc.load_gather(ubuf, [row16, d16])
                xx = plsc.load_gather(xbuf, [row16, d16])
                return a + uu * xx

            def d8_body(d8, a):
                for dd in range(8):
                    a = d_body(d8 * 8 + dd, a)
                return a

            acc = lax.fori_loop(jnp.int32(0), D // 8, d8_body, acc)
            cidx = cu_v[pl.ds(r * GCH + q * 16, 16)]
            s2v = plsc.load_gather(s2_v, [cidx])
            w1v = jnp.float32(EPS) + lr16 * (acc - s2v)
            pos = cp_v[pl.ds(r * GCH + q * 16, 16)]
            plsc.store_scatter(w1_v, [pos], w1v)
            return c2

        lax.fori_loop(jnp.int32(0), GCH // 16, q_body, jnp.int32(0))
        return carry

    lax.fori_loop(jnp.int32(0), nch, gs_body, jnp.int32(0))

    pltpu.sync_copy(w1_v.at[pl.ds(0, SH)], w1_hbm.at[pl.ds(base, SH)])


def _run_kc(u, x, blk, s2, att, lr):
    f = pl.kernel(
        _kc_body,
        out_type=jax.ShapeDtypeStruct((BP,), jnp.float32),
        mesh=_mesh(),
        compiler_params=pltpu.CompilerParams(needs_layout_passes=False,
                                             use_tc_tiling_on_sc=False),
        scratch_types=[
            pltpu.VMEM((CNTSZ,), jnp.int32),
            pltpu.VMEM((CNTSZ,), jnp.float32),
            pltpu.VMEM((IPAD,), jnp.int32),
            pltpu.VMEM((16,), jnp.float32),
            pltpu.VMEM((BCH,), jnp.int32),
            pltpu.VMEM((CSZ,), jnp.int32),
            pltpu.VMEM((CSZ,), jnp.int32),
            pltpu.VMEM((CSZ,), jnp.int32),
            pltpu.VMEM((GCH, D), jnp.float32),
            pltpu.VMEM((GCH, D), jnp.float32),
            pltpu.VMEM((SH + 16,), jnp.float32),
            pltpu.SemaphoreType.DMA,
            pltpu.SemaphoreType.DMA,
        ],
    )
    return f(u, x, blk, s2, att, lr)


def _kd_body(w1_ref, bud_ref, proj_ref, meta_ref):
    w1 = w1_ref[...]
    gid = (lax.broadcasted_iota(jnp.int32, (NW, SH), 0) * SH
           + lax.broadcasted_iota(jnp.int32, (NW, SH), 1))
    valid = gid < B
    budget_f = bud_ref[0, 0]

    s0 = jnp.sum(jnp.where(valid, jnp.clip(w1, 0.0, 1.0), 0.0))
    feasible = s0 <= budget_f
    lo0 = jnp.min(jnp.where(valid, w1, jnp.inf)) - 1.0
    hi0 = jnp.max(jnp.where(valid, w1, -jnp.inf))

    def bis_body(_, carry):
        lo, hi = carry
        mid = (lo + hi) / 2.0
        ex = jnp.sum(jnp.where(valid, jnp.clip(w1 - mid, 0.0, 1.0), 0.0)) \
            - budget_f
        pos = ex > 0
        return jnp.where(pos, mid, lo), jnp.where(pos, hi, mid)

    lo, hi = lax.fori_loop(jnp.int32(0), 40, bis_body, (lo0, hi0))
    mu = (lo + hi) / 2.0
    proj = jnp.where(feasible, jnp.clip(w1, 0.0, 1.0),
                     jnp.clip(w1 - mu, 0.0, 1.0))
    proj = jnp.where(valid, proj, -1.0)
    proj_ref[...] = proj

    def count_gt(thr):
        return jnp.sum((valid & (proj > thr)).astype(jnp.float32))

    kf = jnp.float32(K)
    g0 = count_gt(jnp.float32(0.0))
    maxv = jnp.max(jnp.where(valid, proj, 0.0))
    maxv_i = lax.bitcast_convert_type(maxv, jnp.int32)
    hi_i0 = jnp.where(g0 >= kf, maxv_i, jnp.int32(0))

    def tb_body(_, carry):
        lo_i, hi_i = carry
        mid_i = lax.shift_right_arithmetic(lo_i + hi_i, jnp.int32(1))
        c = count_gt(lax.bitcast_convert_type(mid_i, jnp.float32))
        big = c >= kf
        return jnp.where(big, mid_i, lo_i), jnp.where(big, hi_i, mid_i)

    lo_i, hi_i = lax.fori_loop(jnp.int32(0), 34, tb_body, (jnp.int32(0), hi_i0))
    t = jnp.where(g0 >= kf, lax.bitcast_convert_type(hi_i, jnp.float32),
                  jnp.float32(0.0))
    n_strict = count_gt(t)
    n_tie = kf - n_strict

    sc_w = jnp.sum((valid & (proj > t)).astype(jnp.float32), axis=1,
                   keepdims=True)
    tc_w = jnp.sum((valid & (proj == t)).astype(jnp.float32), axis=1,
                   keepdims=True)
    tri = (lax.broadcasted_iota(jnp.int32, (NW, NW), 0)
           > lax.broadcasted_iota(jnp.int32, (NW, NW), 1)).astype(jnp.float32)
    sp = jnp.dot(tri, sc_w, preferred_element_type=jnp.float32)
    tp = jnp.dot(tri, tc_w, preferred_element_type=jnp.float32)
    nsf = jnp.full((NW, 1), 0.0, jnp.float32) + n_strict
    ntf = jnp.full((NW, 1), 0.0, jnp.float32) + n_tie
    tf = jnp.full((NW, 1), 0.0, jnp.float32) + t
    pad = jnp.zeros((NW, 3), jnp.float32)
    meta_ref[...] = jnp.concatenate([sp, tp, tf, nsf, ntf, pad], axis=1)


def _run_kd(w1p, budf):
    return pl.pallas_call(
        _kd_body,
        out_shape=[
            jax.ShapeDtypeStruct((NW, SH), jnp.float32),
            jax.ShapeDtypeStruct((NW, 8), jnp.float32),
        ],
    )(w1p, budf)


def _ke_body(proj_hbm, metai_hbm, metaf_hbm,
             tv_hbm, ti_hbm,
             pbuf, mi_v, mf_v, vb_v, ibs_v, ibt_v, tvb_v, sem):
    wid = _wid()
    base = pl.multiple_of(wid * SH, 16)

    pltpu.sync_copy(metai_hbm, mi_v)
    pltpu.sync_copy(metaf_hbm, mf_v)
    pltpu.sync_copy(proj_hbm.at[pl.ds(base, SH)], pbuf)

    t16 = mf_v[...]
    vw = mi_v[pl.ds(wid * 8, 16)]
    v0 = mi_v[pl.ds(0, 16)]
    sp_w = vw[0]
    tp_w = vw[1]
    ns_tot = v0[3]
    nt_tot = v0[4]

    def grp_body(k, carry):
        ls, lt = carry
        v16 = pbuf[pl.ds(k * 16, 16)]
        gid = base + k * 16 + lax.iota(jnp.int32, 16)
        ms = v16 > t16
        mt = v16 == t16
        plsc.store_compressed(vb_v.at[pl.ds(ls, 16)], v16, mask=ms)
        plsc.store_compressed(ibs_v.at[pl.ds(ls, 16)], gid, mask=ms)
        plsc.store_compressed(ibt_v.at[pl.ds(lt, 16)], gid, mask=mt)
        return ls + _popcnt(ms), lt + _popcnt(mt)

    ls, lt = lax.fori_loop(jnp.int32(0), SH // 16, grp_body,
                           (jnp.int32(0), jnp.int32(0)))

    lane16 = lax.iota(jnp.int32, 16)
    nch_s = _idiv(ls + GCH - 1, GCH)

    def sc_s(cc, carry):
        j16base = pl.multiple_of(cc * 16, 8)
        j16 = j16base + lane16
        p16 = jnp.where(j16 < ls, sp_w + j16, jnp.int32(OUTP - GCH))
        pltpu.async_copy(vb_v.at[pl.ds(j16base, 16)],
                         tv_hbm.at[p16], sem).wait()
        pltpu.async_copy(ibs_v.at[pl.ds(j16base, 16)],
                         ti_hbm.at[p16], sem).wait()
        return carry

    nch16_s = _idiv(ls + 15, 16)
    lax.fori_loop(jnp.int32(0), nch16_s, sc_s, jnp.int32(0))

    def tv_body(l, carry):
        tvb_v[pl.ds(l * 16, 16)] = t16
        return carry

    lax.fori_loop(jnp.int32(0), 16 // 16, tv_body, jnp.int32(0))
    m_t = jnp.clip(jnp.minimum(lt, nt_tot - tp_w), 0, SH)
    nch16_t = _idiv(m_t + 15, 16)

    def sc_t(cc, carry):
        j16base = pl.multiple_of(cc * 16, 8)
        j16 = j16base + lane16
        tie_g = tp_w + j16
        ok = (j16 < lt) & (tie_g < nt_tot)
        p16 = jnp.where(ok, ns_tot + tie_g, jnp.int32(OUTP - GCH))
        pltpu.async_copy(tvb_v.at[pl.ds(0, 16)], tv_hbm.at[p16], sem).wait()
        pltpu.async_copy(ibt_v.at[pl.ds(j16base, 16)],
                         ti_hbm.at[p16], sem).wait()
        return carry

    lax.fori_loop(jnp.int32(0), nch16_t, sc_t, jnp.int32(0))


def _run_ke(projp, metai, metaf):
    f = pl.kernel(
        _ke_body,
        out_type=(
            jax.ShapeDtypeStruct((OUTP,), jnp.float32),
            jax.ShapeDtypeStruct((OUTP,), jnp.int32),
        ),
        mesh=_mesh(),
        compiler_params=pltpu.CompilerParams(needs_layout_passes=False),
        scratch_types=[
            pltpu.VMEM((SH,), jnp.float32),
            pltpu.VMEM((NW * 8 + 16, ), jnp.int32),
            pltpu.VMEM((16,), jnp.float32),
            pltpu.VMEM((SH + 16,), jnp.float32),
            pltpu.VMEM((SH + 16,), jnp.int32),
            pltpu.VMEM((SH + 16,), jnp.int32),
            pltpu.VMEM((16,), jnp.float32),
            pltpu.SemaphoreType.DMA,
        ],
    )
    return f(projp, metai, metaf)


def _kf_body(vc_ref, vr_ref, ic_ref, ir_ref, ov_ref, oi_ref):
    sc = lax.broadcasted_iota(jnp.int32, (1024, 1), 0)
    sr = lax.broadcasted_iota(jnp.int32, (1, 1024), 1)
    vc = jnp.where(sc < K, vc_ref[...], -2.0)
    vr = jnp.where(sr < K, vr_ref[...], -2.0)
    ic = jnp.where(sc < K, ic_ref[...], 1000000 + sc)
    ir = jnp.where(sr < K, ir_ref[...], 1000000 + sr)
    ahead = (vr > vc) | ((vr == vc) & (ir < ic))
    rank = jnp.sum(ahead.astype(jnp.int32), axis=1, keepdims=True,
                   dtype=jnp.int32)
    eq = rank == sr
    ov_ref[...] = jnp.max(jnp.where(eq, vc, -3.0), axis=0, keepdims=True)
    oi_ref[...] = jnp.max(jnp.where(eq, ic, -1), axis=0, keepdims=True)


def _run_kf(vc, vr, ic, ir):
    return pl.pallas_call(
        _kf_body,
        out_shape=[
            jax.ShapeDtypeStruct((1, 1024), jnp.float32),
            jax.ShapeDtypeStruct((1, 1024), jnp.int32),
        ],
    )(vc, vr, ic, ir)


def kernel(x, edge_index, labels, budget, idx_attack, block, W):
    x = jnp.asarray(x, jnp.float32)
    W = jnp.asarray(W, jnp.float32)
    src = jnp.asarray(edge_index[0], jnp.int32)
    dst = jnp.asarray(edge_index[1], jnp.int32)
    blk = jnp.pad(jnp.asarray(block, jnp.int32), (0, BP - B))
    att = jnp.pad(jnp.asarray(idx_attack, jnp.int32), (0, IPAD - M_ATT),
                  constant_values=N)
    lab = jnp.asarray(labels, jnp.int32).reshape(N, 1)
    budget_f = jnp.asarray(budget).astype(jnp.float32)
    lr_eff = jnp.float32(LR) * budget_f / jnp.float32(N)

    x0 = x[:, :D // 2] + 0.0
    x1 = x[:, D // 2:] + 0.0
    agg2, deg32, cnt = _run_ka(x0, x1, src, dst, blk, att)

    u, s2 = _run_kb(x, agg2[:, :, :N, :], deg32.reshape(NW, N).T,
                    cnt.astype(jnp.float32).reshape(N, 1), lab, W, W.T)

    w1 = _run_kc(u, x, blk, s2[:, 0], att,
                 jnp.broadcast_to(lr_eff, (16,)))

    projp, meta = _run_kd(w1.reshape(NW, SH),
                          jnp.broadcast_to(budget_f, (1, 1)))

    metai = jnp.pad(meta.astype(jnp.int32).reshape(NW * 8), (0, 16))
    metaf = jnp.broadcast_to(meta[0, 2], (16,))
    tv, ti = _run_ke(projp.reshape(BP), metai, metaf)

    ov, oi = _run_kf(tv[:1024].reshape(1024, 1), tv[:1024].reshape(1, 1024),
                     ti[:1024].reshape(1024, 1), ti[:1024].reshape(1, 1024))

    proj = projp.reshape(BP)[:B]
    return proj, ov[0, :K], oi[0, :K].astype(jnp.int32)


# speedup vs baseline: 1.0364x; 1.0364x over previous
import jax
import jax.numpy as jnp
from jax import lax
from jax.experimental import pallas as pl
from jax.experimental.pallas import tpu as pltpu
from jax.experimental.pallas import tpu_sc as plsc

N = 10000
D = 128
E = 320000
C = 32
B = 250000
K = 1000
M_ATT = 1000
EPS = 1e-7
LR = 1000.0

NC = 2
NS = 16
NW = NC * NS
SH = 7824
BP = NW * SH
ECH = 2000
BCH = 2608
GCH = 128
GCA = 128
TRASH = N
NROWS = 10008
IPAD = 1008
CNTSZ = 10016
CSZ = 10384
NCH_MAX = 81
OUTP = 1152


def _mesh():
    return plsc.VectorSubcoreMesh(core_axis_name="c", subcore_axis_name="s")


def _wid():
    return lax.axis_index("s") * NC + lax.axis_index("c")


def _idiv(a, b):
    return lax.div(a, jnp.int32(b))


def _irem(a, b):
    return lax.rem(a, jnp.int32(b))


def _popcnt(m):
    return plsc.all_reduce_population_count(m)[0]


def _memset_i32(ref, size, value):
    v = jnp.full((16,), value, jnp.int32)

    def body(i, carry):
        ref[pl.ds(i * 16, 16)] = v
        return carry

    lax.fori_loop(jnp.int32(0), size // 16, body, jnp.int32(0))


def _memset_f32(ref, size, value):
    v = jnp.full((16,), value, jnp.float32)

    def body(i, carry):
        ref[pl.ds(i * 16, 16)] = v
        return carry

    lax.fori_loop(jnp.int32(0), size // 16, body, jnp.int32(0))


def _build_cnt(cnt_ref, att_ref):
    _memset_i32(cnt_ref, CNTSZ, 0)
    ones = jnp.ones((16,), jnp.int32)

    def body(i, carry):
        idx = att_ref[pl.ds(i * 16, 16)]
        plsc.addupdate_scatter(cnt_ref, [idx], ones)
        return carry

    lax.fori_loop(jnp.int32(0), IPAD // 16, body, jnp.int32(0))


def _ka_body(x0_hbm, x1_hbm, src_hbm, dst_hbm, blk_hbm, att_hbm,
             agg_hbm, deg_hbm, cnt_hbm,
             cnt_v, deg_v, att_v, sbuf_a, sbuf_b,
             csrc_v, cdst_v, rows_v, rows2_v, agg_sh, sem, sem2):
    wid = _wid()
    cid = lax.axis_index("c")
    sid = lax.axis_index("s")

    pltpu.sync_copy(att_hbm, att_v)
    _build_cnt(cnt_v, att_v)
    _memset_f32(deg_v, CNTSZ, 0.0)

    def zero_rows():
        def zz_body(i, carry):
            rows_v[_idiv(i, 4), pl.ds(_irem(i, 4) * 16, 16)] = \
                jnp.zeros((16,), jnp.float32)
            return carry

        lax.fori_loop(jnp.int32(0), GCA * (D // 32), zz_body, jnp.int32(0))

    r0 = sid * 640
    starts = [0, GCA, 2 * GCA, 3 * GCA, 640 - GCA]

    def sp_start(j):
        s = r0 + starts[j]
        return pl.multiple_of(jnp.minimum(s, NROWS - GCA), 8)

    def zero_agg():
        zero_rows()
        for j in range(len(starts)):
            pltpu.sync_copy(rows_v, agg_sh.at[pl.ds(sp_start(j), GCA)])

    def process_edges(n_edges, ech, base_edge, is_block, half, do_deg):
        _memset_i32(csrc_v, CSZ, 0)
        _memset_i32(cdst_v, CSZ, TRASH)

        def stage_body(s, nsurv):
            sb = pl.multiple_of(base_edge + s * ech, 16)
            if is_block:
                pltpu.sync_copy(blk_hbm.at[pl.ds(sb, ech)],
                                sbuf_a.at[pl.ds(0, ech)])
            else:
                pltpu.sync_copy(src_hbm.at[pl.ds(sb, ech)],
                                sbuf_a.at[pl.ds(0, ech)])
                pltpu.sync_copy(dst_hbm.at[pl.ds(sb, ech)],
                                sbuf_b.at[pl.ds(0, ech)])

            def grp_body(k, off):
                if is_block:
                    b16 = sbuf_a[pl.ds(k * 16, 16)]
                    r16 = _idiv(b16, N)
                    c16 = _irem(b16, N)
                    c16 = jnp.where(r16 == c16, _irem(c16 + 1, N), c16)
                    gid = sb + k * 16 + lax.iota(jnp.int32, 16)
                    valid = gid < B
                    s16, d16 = r16, c16
                    w16 = jnp.where(valid, jnp.float32(EPS), 0.0)
                else:
                    s16 = sbuf_a[pl.ds(k * 16, 16)]
                    d16 = sbuf_b[pl.ds(k * 16, 16)]
                    valid = d16 >= 0
                    w16 = jnp.ones((16,), jnp.float32)
                if do_deg:
                    plsc.addupdate_scatter(deg_v, [d16], w16)
                m = (plsc.load_gather(cnt_v, [d16]) > 0) & valid
                plsc.store_compressed(csrc_v.at[pl.ds(off, 16)], s16, mask=m)
                plsc.store_compressed(cdst_v.at[pl.ds(off, 16)], d16, mask=m)
                return off + _popcnt(m)

            return lax.fori_loop(jnp.int32(0), ech // 16, grp_body, nsurv)

        nsurv = lax.fori_loop(jnp.int32(0), n_edges // ech, stage_body, jnp.int32(0))

        nch = _idiv(nsurv + GCA - 1, GCA)

        xh_hbm = x1_hbm if half else x0_hbm
        bufs = (rows_v, rows2_v)
        sems = (sem, sem2)

        def fire(r, b):
            rg = pl.multiple_of(r * GCA, 8)
            return pltpu.async_copy(
                xh_hbm.at[csrc_v.at[pl.ds(rg, GCA)]], bufs[b], sems[b])

        def drain(r, b):
            rg = pl.multiple_of(r * GCA, 8)
            buf = bufs[b]
            if is_block:
                def sc_body(q, c2):
                    qr = _idiv(q, 4)
                    qc = _irem(q, 4) * 16
                    v = buf[qr, pl.ds(qc, 16)]
                    buf[qr, pl.ds(qc, 16)] = v * jnp.float32(EPS)
                    return c2

                lax.fori_loop(jnp.int32(0), GCA * (D // 32), sc_body,
                              jnp.int32(0))
            descs = []
            for sub in range(GCA // 16):
                idx16 = cdst_v[pl.ds(rg + sub * 16, 16)]
                descs.append(pltpu.async_copy(
                    buf.at[pl.ds(sub * 16, 16)], agg_sh.at[idx16],
                    sems[b], add=True))
            for dsc in descs:
                dsc.wait()

        def wait_g(r, b):
            rg = pl.multiple_of(r * GCA, 8)
            pltpu.make_async_copy(
                xh_hbm.at[csrc_v.at[pl.ds(rg, GCA)]], bufs[b],
                sems[b]).wait()

        @pl.when(nch > 0)
        def _():
            fire(jnp.int32(0), 0)

        def pair_body(pp, carry):
            r0 = pp * 2
            r1 = r0 + 1

            @pl.when(r1 < nch)
            def _():
                fire(r1, 1)

            wait_g(r0, 0)
            drain(r0, 0)

            @pl.when(r0 + 2 < nch)
            def _():
                fire(r0 + 2, 0)

            @pl.when(r1 < nch)
            def _():
                wait_g(r1, 1)
                drain(r1, 1)
            return carry

        lax.fori_loop(jnp.int32(0), _idiv(nch + 1, 2), pair_body,
                      jnp.int32(0))

    for h in range(2):
        zero_agg()
        plsc.subcore_barrier()
        process_edges(E // NW, ECH, wid * (E // NW), False, h, h == 0)
        process_edges(SH, BCH, wid * SH, True, h, h == 0)
        plsc.subcore_barrier()
        for j in range(len(starts)):
            sj = sp_start(j)
            pltpu.sync_copy(agg_sh.at[pl.ds(sj, GCA)], rows_v)
            pltpu.sync_copy(rows_v,
                            agg_hbm.at[cid, jnp.int32(h), pl.ds(sj, GCA)])
        plsc.subcore_barrier()

    pltpu.sync_copy(deg_v.at[pl.ds(0, N)],
                    deg_hbm.at[pl.ds(pl.multiple_of(wid * N, 8), N)])

    @pl.when(wid == 0)
    def _():
        pltpu.sync_copy(cnt_v.at[pl.ds(0, N)], cnt_hbm)


def _run_ka(x0, x1, src, dst, blk, att):
    f = pl.kernel(
        _ka_body,
        out_type=(
            jax.ShapeDtypeStruct((NC, 2, NROWS, D // 2), jnp.float32),
            jax.ShapeDtypeStruct((NW * N,), jnp.float32),
            jax.ShapeDtypeStruct((N,), jnp.int32),
        ),
        mesh=_mesh(),
        compiler_params=pltpu.CompilerParams(needs_layout_passes=False,
                                             use_tc_tiling_on_sc=False),
        scratch_types=[
            pltpu.VMEM((CNTSZ,), jnp.int32),
            pltpu.VMEM((CNTSZ,), jnp.float32),
            pltpu.VMEM((IPAD,), jnp.int32),
            pltpu.VMEM((BCH,), jnp.int32),
            pltpu.VMEM((BCH,), jnp.int32),
            pltpu.VMEM((CSZ,), jnp.int32),
            pltpu.VMEM((CSZ,), jnp.int32),
            pltpu.VMEM((GCA, D // 2), jnp.float32),
            pltpu.VMEM((GCA, D // 2), jnp.float32),
            pltpu.VMEM_SHARED((NROWS, D // 2), jnp.float32),
            pltpu.SemaphoreType.DMA,
            pltpu.SemaphoreType.DMA,
        ],
    )
    return f(x0, x1, src, dst, blk, att)


def _kb_body(x_ref, agg_ref, deg_ref, cntf_ref, lab_ref, w_ref, wt_ref,
             u_ref, s2_ref):
    x = x_ref[...]
    agg = jnp.concatenate(
        [agg_ref[0, 0] + agg_ref[1, 0], agg_ref[0, 1] + agg_ref[1, 1]],
        axis=-1)
    deg = jnp.sum(deg_ref[...], axis=1) + 1.0
    h = (agg + x) / deg[:, None]
    logits = jnp.dot(h, w_ref[...], preferred_element_type=jnp.float32)
    mx = jnp.max(logits, axis=-1, keepdims=True)
    ex = jnp.exp(logits - mx)
    p = ex / jnp.sum(ex, axis=-1, keepdims=True)
    lab = lab_ref[...]
    oh = lab == lax.broadcasted_iota(jnp.int32, (1, C), 1)
    pm = jnp.where(oh, -jnp.inf, p)
    bo = jnp.max(pm, axis=-1, keepdims=True)
    bsel = jnp.logical_and(jnp.logical_not(oh), p == bo)
    nb = jnp.maximum(
        jnp.sum(bsel.astype(jnp.float32), axis=-1, keepdims=True), 1.0)
    cnt = cntf_ref[...]
    dl_p = (cnt / jnp.float32(M_ATT)) * (bsel.astype(jnp.float32) / nb
                                         - oh.astype(jnp.float32))
    dlg = p * (dl_p - jnp.sum(dl_p * p, axis=-1, keepdims=True))
    dh = jnp.dot(dlg, wt_ref[...], preferred_element_type=jnp.float32)
    u = dh / deg[:, None]
    u_ref[...] = u
    s2_ref[...] = jnp.sum(u * h, axis=-1, keepdims=True)


def _run_kb(x, agg2, deg32, cntf, lab, W, WT):
    blk = 1000
    z = lambda i: i * 0
    return pl.pallas_call(
        _kb_body,
        grid=(N // blk,),
        in_specs=[
            pl.BlockSpec((blk, D), lambda i: (i, z(i))),
            pl.BlockSpec((NC, 2, blk, D // 2),
                         lambda i: (z(i), z(i), i, z(i))),
            pl.BlockSpec((blk, NW), lambda i: (i, z(i))),
            pl.BlockSpec((blk, 1), lambda i: (i, z(i))),
            pl.BlockSpec((blk, 1), lambda i: (i, z(i))),
            pl.BlockSpec((D, C), lambda i: (z(i), z(i))),
            pl.BlockSpec((C, D), lambda i: (z(i), z(i))),
        ],
        out_specs=[
            pl.BlockSpec((blk, D), lambda i: (i, z(i))),
            pl.BlockSpec((blk, 1), lambda i: (i, z(i))),
        ],
        out_shape=[
            jax.ShapeDtypeStruct((N, D), jnp.float32),
            jax.ShapeDtypeStruct((N, 1), jnp.float32),
        ],
    )(x, agg2, deg32, cntf, lab, W, WT)


def _kc_body(u_hbm, x_hbm, blk_hbm, s2_hbm, att_hbm, lr_hbm,
             w1_hbm,
             cnt_v, s2_v, att_v, lr_v, bbuf,
             cu_v, cx_v, cp_v, ubuf, xbuf, w1_v, sem, sem2):
    wid = _wid()
    base = pl.multiple_of(wid * SH, 16)

    pltpu.sync_copy(att_hbm, att_v)
    _build_cnt(cnt_v, att_v)
    pltpu.sync_copy(s2_hbm, s2_v.at[pl.ds(0, N)])
    pltpu.sync_copy(lr_hbm, lr_v)
    lr16 = lr_v[...]

    def init_body(k, carry):
        gid = base + k * 16 + lax.iota(jnp.int32, 16)
        w1_v[pl.ds(k * 16, 16)] = jnp.where(gid < B, jnp.float32(EPS), -1.0)
        return carry

    lax.fori_loop(jnp.int32(0), SH // 16, init_body, jnp.int32(0))

    _memset_i32(cu_v, CSZ, 0)
    _memset_i32(cx_v, CSZ, 0)
    _memset_i32(cp_v, CSZ, SH)

    def stage_body(s, nsurv):
        sb = s * BCH
        bs = pl.multiple_of(base + sb, 16)
        pltpu.sync_copy(blk_hbm.at[pl.ds(bs, BCH)], bbuf)

        def grp_body(k, off):
            b16 = bbuf[pl.ds(k * 16, 16)]
            r16 = _idiv(b16, N)
            c16 = _irem(b16, N)
            c16 = jnp.where(r16 == c16, _irem(c16 + 1, N), c16)
            lpos = sb + k * 16 + lax.iota(jnp.int32, 16)
            gid = base + lpos
            m = (plsc.load_gather(cnt_v, [c16]) > 0) & (gid < B)
            plsc.store_compressed(cu_v.at[pl.ds(off, 16)], c16, mask=m)
            plsc.store_compressed(cx_v.at[pl.ds(off, 16)], r16, mask=m)
            plsc.store_compressed(cp_v.at[pl.ds(off, 16)], lpos, mask=m)
            return off + _popcnt(m)

        return lax.fori_loop(jnp.int32(0), BCH // 16, grp_body, nsurv)

    nsurv = lax.fori_loop(jnp.int32(0), SH // BCH, stage_body, jnp.int32(0))

    nch = _idiv(nsurv + GCH - 1, GCH)
    lane16 = lax.iota(jnp.int32, 16)

    def gs_body(r, carry):
        rg = pl.multiple_of(r * GCH, 8)
        du = pltpu.async_copy(u_hbm.at[cu_v.at[pl.ds(rg, GCH)]], ubuf, sem)
        dx = pltpu.async_copy(x_hbm.at[cx_v.at[pl.ds(rg, GCH)]], xbuf, sem2)
        du.wait()
        dx.wait()

        def q_body(q, c2):
            row16 = q * 16 + lane16
            acc = jnp.zeros((16,), jnp.float32)

            def d_body(d, a):
                d16 = jnp.full((16,), 0, jnp.int32) + d
                uu = plsc.load_gather(ubuf, [row16, d16])
                xx = plsc.load_gather(xbuf, [row16, d16])
                return a + uu * xx

            def d8_body(d8, a):
                for dd in range(8):
                    a = d_body(d8 * 8 + dd, a)
                return a

            acc = lax.fori_loop(jnp.int32(0), D // 8, d8_body, acc)
            cidx = cu_v[pl.ds(r * GCH + q * 16, 16)]
            s2v = plsc.load_gather(s2_v, [cidx])
            w1v = jnp.float32(EPS) + lr16 * (acc - s2v)
            pos = cp_v[pl.ds(r * GCH + q * 16, 16)]
            plsc.store_scatter(w1_v, [pos], w1v)
            return c2

        lax.fori_loop(jnp.int32(0), GCH // 16, q_body, jnp.int32(0))
        return carry

    lax.fori_loop(jnp.int32(0), nch, gs_body, jnp.int32(0))

    pltpu.sync_copy(w1_v.at[pl.ds(0, SH)], w1_hbm.at[pl.ds(base, SH)])


def _run_kc(u, x, blk, s2, att, lr):
    f = pl.kernel(
        _kc_body,
        out_type=jax.ShapeDtypeStruct((BP,), jnp.float32),
        mesh=_mesh(),
        compiler_params=pltpu.CompilerParams(needs_layout_passes=False,
                                             use_tc_tiling_on_sc=False),
        scratch_types=[
            pltpu.VMEM((CNTSZ,), jnp.int32),
            pltpu.VMEM((CNTSZ,), jnp.float32),
            pltpu.VMEM((IPAD,), jnp.int32),
            pltpu.VMEM((16,), jnp.float32),
            pltpu.VMEM((BCH,), jnp.int32),
            pltpu.VMEM((CSZ,), jnp.int32),
            pltpu.VMEM((CSZ,), jnp.int32),
            pltpu.VMEM((CSZ,), jnp.int32),
            pltpu.VMEM((GCH, D), jnp.float32),
            pltpu.VMEM((GCH, D), jnp.float32),
            pltpu.VMEM((SH + 16,), jnp.float32),
            pltpu.SemaphoreType.DMA,
            pltpu.SemaphoreType.DMA,
        ],
    )
    return f(u, x, blk, s2, att, lr)


def _kd_body(w1_ref, bud_ref, proj_ref, meta_ref):
    w1 = w1_ref[...]
    gid = (lax.broadcasted_iota(jnp.int32, (NW, SH), 0) * SH
           + lax.broadcasted_iota(jnp.int32, (NW, SH), 1))
    valid = gid < B
    budget_f = bud_ref[0, 0]

    s0 = jnp.sum(jnp.where(valid, jnp.clip(w1, 0.0, 1.0), 0.0))
    feasible = s0 <= budget_f
    lo0 = jnp.min(jnp.where(valid, w1, jnp.inf)) - 1.0
    hi0 = jnp.max(jnp.where(valid, w1, -jnp.inf))

    def bis_body(_, carry):
        lo, hi = carry
        mid = (lo + hi) / 2.0
        ex = jnp.sum(jnp.where(valid, jnp.clip(w1 - mid, 0.0, 1.0), 0.0)) \
            - budget_f
        pos = ex > 0
        return jnp.where(pos, mid, lo), jnp.where(pos, hi, mid)

    lo, hi = lax.fori_loop(jnp.int32(0), 40, bis_body, (lo0, hi0))
    mu = (lo + hi) / 2.0
    proj = jnp.where(feasible, jnp.clip(w1, 0.0, 1.0),
                     jnp.clip(w1 - mu, 0.0, 1.0))
    proj = jnp.where(valid, proj, -1.0)
    proj_ref[...] = proj

    def count_gt(thr):
        return jnp.sum((valid & (proj > thr)).astype(jnp.float32))

    kf = jnp.float32(K)
    g0 = count_gt(jnp.float32(0.0))
    maxv = jnp.max(jnp.where(valid, proj, 0.0))
    maxv_i = lax.bitcast_convert_type(maxv, jnp.int32)
    hi_i0 = jnp.where(g0 >= kf, maxv_i, jnp.int32(0))

    def tb_body(_, carry):
        lo_i, hi_i = carry
        mid_i = lax.shift_right_arithmetic(lo_i + hi_i, jnp.int32(1))
        c = count_gt(lax.bitcast_convert_type(mid_i, jnp.float32))
        big = c >= kf
        return jnp.where(big, mid_i, lo_i), jnp.where(big, hi_i, mid_i)

    lo_i, hi_i = lax.fori_loop(jnp.int32(0), 34, tb_body, (jnp.int32(0), hi_i0))
    t = jnp.where(g0 >= kf, lax.bitcast_convert_type(hi_i, jnp.float32),
                  jnp.float32(0.0))
    n_strict = count_gt(t)
    n_tie = kf - n_strict

    sc_w = jnp.sum((valid & (proj > t)).astype(jnp.float32), axis=1,
                   keepdims=True)
    tc_w = jnp.sum((valid & (proj == t)).astype(jnp.float32), axis=1,
                   keepdims=True)
    tri = (lax.broadcasted_iota(jnp.int32, (NW, NW), 0)
           > lax.broadcasted_iota(jnp.int32, (NW, NW), 1)).astype(jnp.float32)
    sp = jnp.dot(tri, sc_w, preferred_element_type=jnp.float32)
    tp = jnp.dot(tri, tc_w, preferred_element_type=jnp.float32)
    nsf = jnp.full((NW, 1), 0.0, jnp.float32) + n_strict
    ntf = jnp.full((NW, 1), 0.0, jnp.float32) + n_tie
    tf = jnp.full((NW, 1), 0.0, jnp.float32) + t
    pad = jnp.zeros((NW, 3), jnp.float32)
    meta_ref[...] = jnp.concatenate([sp, tp, tf, nsf, ntf, pad], axis=1)


def _run_kd(w1p, budf):
    return pl.pallas_call(
        _kd_body,
        out_shape=[
            jax.ShapeDtypeStruct((NW, SH), jnp.float32),
            jax.ShapeDtypeStruct((NW, 8), jnp.float32),
        ],
    )(w1p, budf)


def _ke_body(proj_hbm, metai_hbm, metaf_hbm,
             tv_hbm, ti_hbm,
             pbuf, mi_v, mf_v, vb_v, ibs_v, ibt_v, tvb_v, sem):
    wid = _wid()
    base = pl.multiple_of(wid * SH, 16)

    pltpu.sync_copy(metai_hbm, mi_v)
    pltpu.sync_copy(metaf_hbm, mf_v)
    pltpu.sync_copy(proj_hbm.at[pl.ds(base, SH)], pbuf)

    t16 = mf_v[...]
    vw = mi_v[pl.ds(wid * 8, 16)]
    v0 = mi_v[pl.ds(0, 16)]
    sp_w = vw[0]
    tp_w = vw[1]
    ns_tot = v0[3]
    nt_tot = v0[4]

    def grp_body(k, carry):
        ls, lt = carry
        v16 = pbuf[pl.ds(k * 16, 16)]
        gid = base + k * 16 + lax.iota(jnp.int32, 16)
        ms = v16 > t16
        mt = v16 == t16
        plsc.store_compressed(vb_v.at[pl.ds(ls, 16)], v16, mask=ms)
        plsc.store_compressed(ibs_v.at[pl.ds(ls, 16)], gid, mask=ms)
        plsc.store_compressed(ibt_v.at[pl.ds(lt, 16)], gid, mask=mt)
        return ls + _popcnt(ms), lt + _popcnt(mt)

    ls, lt = lax.fori_loop(jnp.int32(0), SH // 16, grp_body,
                           (jnp.int32(0), jnp.int32(0)))

    lane16 = lax.iota(jnp.int32, 16)
    nch_s = _idiv(ls + GCH - 1, GCH)

    def sc_s(cc, carry):
        j16base = pl.multiple_of(cc * 16, 8)
        j16 = j16base + lane16
        p16 = jnp.where(j16 < ls, sp_w + j16, jnp.int32(OUTP - GCH))
        pltpu.async_copy(vb_v.at[pl.ds(j16base, 16)],
                         tv_hbm.at[p16], sem).wait()
        pltpu.async_copy(ibs_v.at[pl.ds(j16base, 16)],
                         ti_hbm.at[p16], sem).wait()
        return carry

    nch16_s = _idiv(ls + 15, 16)
    lax.fori_loop(jnp.int32(0), nch16_s, sc_s, jnp.int32(0))

    def tv_body(l, carry):
        tvb_v[pl.ds(l * 16, 16)] = t16
        return carry

    lax.fori_loop(jnp.int32(0), 16 // 16, tv_body, jnp.int32(0))
    m_t = jnp.clip(jnp.minimum(lt, nt_tot - tp_w), 0, SH)
    nch16_t = _idiv(m_t + 15, 16)

    def sc_t(cc, carry):
        j16base = pl.multiple_of(cc * 16, 8)
        j16 = j16base + lane16
        tie_g = tp_w + j16
        ok = (j16 < lt) & (tie_g < nt_tot)
        p16 = jnp.where(ok, ns_tot + tie_g, jnp.int32(OUTP - GCH))
        pltpu.async_copy(tvb_v.at[pl.ds(0, 16)], tv_hbm.at[p16], sem).wait()
        pltpu.async_copy(ibt_v.at[pl.ds(j16base, 16)],
                         ti_hbm.at[p16], sem).wait()
        return carry

    lax.fori_loop(jnp.int32(0), nch16_t, sc_t, jnp.int32(0))


def _run_ke(projp, metai, metaf):
    f = pl.kernel(
        _ke_body,
        out_type=(
            jax.ShapeDtypeStruct((OUTP,), jnp.float32),
            jax.ShapeDtypeStruct((OUTP,), jnp.int32),
        ),
        mesh=_mesh(),
        compiler_params=pltpu.CompilerParams(needs_layout_passes=False),
        scratch_types=[
            pltpu.VMEM((SH,), jnp.float32),
            pltpu.VMEM((NW * 8 + 16, ), jnp.int32),
            pltpu.VMEM((16,), jnp.float32),
            pltpu.VMEM((SH + 16,), jnp.float32),
            pltpu.VMEM((SH + 16,), jnp.int32),
            pltpu.VMEM((SH + 16,), jnp.int32),
            pltpu.VMEM((16,), jnp.float32),
            pltpu.SemaphoreType.DMA,
        ],
    )
    return f(projp, metai, metaf)


def _kf_body(vc_ref, vr_ref, ic_ref, ir_ref, ov_ref, oi_ref):
    sc = lax.broadcasted_iota(jnp.int32, (1024, 1), 0)
    sr = lax.broadcasted_iota(jnp.int32, (1, 1024), 1)
    vc = jnp.where(sc < K, vc_ref[...], -2.0)
    vr = jnp.where(sr < K, vr_ref[...], -2.0)
    ic = jnp.where(sc < K, ic_ref[...], 1000000 + sc)
    ir = jnp.where(sr < K, ir_ref[...], 1000000 + sr)
    ahead = (vr > vc) | ((vr == vc) & (ir < ic))
    rank = jnp.sum(ahead.astype(jnp.int32), axis=1, keepdims=True,
                   dtype=jnp.int32)
    eq = rank == sr
    ov_ref[...] = jnp.max(jnp.where(eq, vc, -3.0), axis=0, keepdims=True)
    oi_ref[...] = jnp.max(jnp.where(eq, ic, -1), axis=0, keepdims=True)


def _run_kf(vc, vr, ic, ir):
    return pl.pallas_call(
        _kf_body,
        out_shape=[
            jax.ShapeDtypeStruct((1, 1024), jnp.float32),
            jax.ShapeDtypeStruct((1, 1024), jnp.int32),
        ],
    )(vc, vr, ic, ir)


def kernel(x, edge_index, labels, budget, idx_attack, block, W):
    x = jnp.asarray(x, jnp.float32)
    W = jnp.asarray(W, jnp.float32)
    src = jnp.asarray(edge_index[0], jnp.int32)
    dst = jnp.asarray(edge_index[1], jnp.int32)
    blk = jnp.pad(jnp.asarray(block, jnp.int32), (0, BP - B))
    att = jnp.pad(jnp.asarray(idx_attack, jnp.int32), (0, IPAD - M_ATT),
                  constant_values=N)
    lab = jnp.asarray(labels, jnp.int32).reshape(N, 1)
    budget_f = jnp.asarray(budget).astype(jnp.float32)
    lr_eff = jnp.float32(LR) * budget_f / jnp.float32(N)

    x0 = x[:, :D // 2] + 0.0
    x1 = x[:, D // 2:] + 0.0
    agg2, deg32, cnt = _run_ka(x0, x1, src, dst, blk, att)

    u, s2 = _run_kb(x, agg2[:, :, :N, :], deg32.reshape(NW, N).T,
                    cnt.astype(jnp.float32).reshape(N, 1), lab, W, W.T)

    w1 = _run_kc(u, x, blk, s2[:, 0], att,
                 jnp.broadcast_to(lr_eff, (16,)))

    projp, meta = _run_kd(w1.reshape(NW, SH),
                          jnp.broadcast_to(budget_f, (1, 1)))

    metai = jnp.pad(meta.astype(jnp.int32).reshape(NW * 8), (0, 16))
    metaf = jnp.broadcast_to(meta[0, 2], (16,))
    tv, ti = _run_ke(projp.reshape(BP), metai, metaf)

    ov, oi = _run_kf(tv[:1024].reshape(1024, 1), tv[:1024].reshape(1, 1024),
                     ti[:1024].reshape(1024, 1), ti[:1024].reshape(1, 1024))

    proj = projp.reshape(BP)[:B]
    return proj, ov[0, :K], oi[0, :K].astype(jnp.int32)
